# baseline (speedup 1.0000x reference)
import functools

import jax
import jax.numpy as jnp
from jax import lax
from jax.experimental import pallas as pl
from jax.experimental.pallas import tpu as pltpu
from jax.experimental.pallas import tpu_sc as plsc

N = 100000
D = 128
H = 64
G = 256


BLK = 10000


def _gate_body(xa_ref, xb_ref, w1_ref, b1_ref, w2_ref, b2_ref, gate_ref):
    def gate_of(x_blk):
        h = jnp.dot(x_blk, w1_ref[...], preferred_element_type=jnp.float32)
        h = jnp.maximum(h + b1_ref[...], 0.0)
        return jnp.sum(h * w2_ref[...], axis=1, keepdims=True) + b2_ref[...]

    gate_ref[...] = jnp.concatenate([gate_of(xa_ref[...]), gate_of(xb_ref[...])], axis=0)


def _gate_mlp(x, W1, b1r, w2r, b2r):
    return pl.pallas_call(
        _gate_body,
        grid=(N // (2 * BLK),),
        in_specs=[
            pl.BlockSpec((BLK, D), lambda i: (2 * i, 0)),
            pl.BlockSpec((BLK, D), lambda i: (2 * i + 1, 0)),
            pl.BlockSpec((D, H), lambda i: (0, 0)),
            pl.BlockSpec((1, H), lambda i: (0, 0)),
            pl.BlockSpec((1, H), lambda i: (0, 0)),
            pl.BlockSpec((1, 1), lambda i: (0, 0)),
        ],
        out_specs=pl.BlockSpec((2 * BLK, 1), lambda i: (i, 0)),
        out_shape=jax.ShapeDtypeStruct((N, 1), jnp.float32),
    )(x, x, W1, b1r, w2r, b2r)


_NC = 2
_NS = 16
_L = 16
_NW = _NC * _NS
_SEG_PER_W = G // _NW
_CAPW = 8192
_CG = 512
_CX = 256


def _red16(v, op):
    r = v[0]
    for i in range(1, _L):
        r = op(r, v[i])
    return r


def _vrecip(den):
    return jnp.ones((_L,), dtype=jnp.float32) / (
        jnp.zeros((_L,), dtype=jnp.float32) + den)


def _seg_kernel_body(gate_hbm, batch_hbm, x_hbm, out_hbm,
                     vals_v, gbig_v, e_v, x0_v, x1_v, acc_area, dinv_v,
                     sem, semx0, semx1):
    wid = lax.axis_index("s") * _NC + lax.axis_index("c")
    lane = lax.iota(jnp.int32, _L)
    g0 = wid * _SEG_PER_W

    q = g0 + lane

    def bs_step(t, lohi):
        lo, hi = lohi
        mid = (lo + hi) >> 1
        pltpu.async_copy(batch_hbm.at[jnp.minimum(mid, N - 1)], vals_v, sem).wait()
        lt = vals_v[...] < q
        upd = lo < hi
        lo2 = jnp.where(upd, jnp.where(lt, mid + 1, lo), lo)
        hi2 = jnp.where(upd, jnp.where(lt, hi, mid), hi)
        return (lo2, hi2)

    swin, _ = lax.fori_loop(0, 17, bs_step,
                            (jnp.zeros((_L,), jnp.int32),
                             jnp.full((_L,), N, dtype=jnp.int32)))
    sb = [swin[i] for i in range(_SEG_PER_W + 1)]
    w_s, w_e = sb[0], sb[_SEG_PER_W]
    w_b0 = jnp.minimum((w_s // 8) * 8, N - _CAPW)
    fits = (w_e - w_b0) <= _CAPW

    acc0 = tuple(jnp.zeros((_L,), dtype=jnp.float32) for _ in range(D // _L))
    zero16 = jnp.zeros((_L,), dtype=jnp.float32)

    def fast_path():
        pltpu.sync_copy(gate_hbm.at[pl.ds(w_b0, _CAPW)], gbig_v.at[pl.ds(0, _CAPW)])
        nwg = (w_e - w_b0 + _L - 1) // _L

        def zero_grp(j, _):
            e_v[pl.ds(j * _L, _L)] = zero16
            return 0

        lax.fori_loop(0, nwg, zero_grp, 0)

        for sloc in range(_SEG_PER_W):
            s_g, e_g = sb[sloc], sb[sloc + 1]
            j0 = (s_g - w_b0) // _L
            j1 = (e_g - w_b0 + _L - 1) // _L

            def grp0(j, mv, s_g=s_g, e_g=e_g):
                idx = w_b0 + j * _L + lane
                v = gbig_v[pl.ds(j * _L, _L)]
                msk = (idx >= s_g) & (idx < e_g)
                return jnp.maximum(mv, jnp.where(msk, v, -jnp.inf))

            m_vec = lax.fori_loop(j0, j1, grp0,
                                  jnp.full((_L,), -jnp.inf, dtype=jnp.float32))
            m = _red16(m_vec, jnp.maximum)

            def grp1(j, dv, s_g=s_g, e_g=e_g, m=m):
                idx = w_b0 + j * _L + lane
                v = gbig_v[pl.ds(j * _L, _L)]
                msk = (idx >= s_g) & (idx < e_g)
                ev = jnp.where(msk, jnp.exp(v - m), 0.0)
                e_v[pl.ds(j * _L, _L)] = e_v[pl.ds(j * _L, _L)] + ev
                return dv + ev

            d_vec = lax.fori_loop(j0, j1, grp1, jnp.zeros((_L,), dtype=jnp.float32))
            dinv_v[pl.ds(sloc * _L, _L)] = _vrecip(_red16(d_vec, jnp.add) + 1e-16)

        for sloc in range(_SEG_PER_W):
            for c in range(D // _L):
                acc_area[sloc, pl.ds(c * _L, _L)] = zero16

        nch = (w_e - w_b0 + _CX - 1) // _CX
        npairs = (nch + 1) // 2

        def xbase(k):
            return jnp.minimum(w_b0 + k * _CX, N - _CX)

        def process_chunk(k, xbuf):
            b = xbase(k)
            rel = b - w_b0
            for sloc in range(_SEG_PER_W):
                s_g, e_g = sb[sloc], sb[sloc + 1]
                glo = jnp.maximum(s_g, b)
                ghi = jnp.minimum(e_g, b + _CX)
                cj0 = jnp.maximum(glo - b, 0) // _L
                cj1 = (jnp.maximum(ghi - b, 0) + _L - 1) // _L
                dinv_g = dinv_v[pl.ds(sloc * _L, _L)]

                def grp(j, accs, s_g=s_g, e_g=e_g, b=b, rel=rel, dinv_g=dinv_g):
                    idx = b + j * _L + lane
                    ev = e_v[pl.ds(rel + j * _L, _L)]
                    msk = (idx >= s_g) & (idx < e_g)
                    a = jnp.where(msk, ev, 0.0) * dinv_g
                    acc_out = list(accs)
                    for r in range(_L):
                        ar = a[r]
                        for c in range(D // _L):
                            acc_out[c] = acc_out[c] + ar * xbuf[j * _L + r, pl.ds(c * _L, _L)]
                    return tuple(acc_out)

                accs_in = tuple(acc_area[sloc, pl.ds(c * _L, _L)] for c in range(D // _L))
                accs = lax.fori_loop(cj0, cj1, grp, accs_in)
                for c in range(D // _L):
                    acc_area[sloc, pl.ds(c * _L, _L)] = accs[c]

        pltpu.async_copy(x_hbm.at[pl.ds(xbase(0), _CX)], x0_v, semx0)

        def pair_body(p, _):
            pltpu.async_copy(x_hbm.at[pl.ds(xbase(2 * p + 1), _CX)], x1_v, semx1)
            pltpu.make_async_copy(x_hbm.at[pl.ds(0, _CX)], x0_v, semx0).wait()
            process_chunk(2 * p, x0_v)
            pltpu.async_copy(x_hbm.at[pl.ds(xbase(2 * p + 2), _CX)], x0_v, semx0)
            pltpu.make_async_copy(x_hbm.at[pl.ds(0, _CX)], x1_v, semx1).wait()
            process_chunk(2 * p + 1, x1_v)
            return 0

        lax.fori_loop(0, npairs, pair_body, 0)
        pltpu.make_async_copy(x_hbm.at[pl.ds(0, _CX)], x0_v, semx0).wait()
        return 0

    def slow_path():
        def seg_body(sloc, _):
            s = _red16(jnp.where(lane == sloc, swin, 0), jnp.add)
            e = _red16(jnp.where(lane == sloc + 1, swin, 0), jnp.add)
            c0 = (s // 8) * 8
            nch_g = (e - c0 + _CG - 1) // _CG

            def max_chunk(k, m_vec):
                b = jnp.minimum(c0 + k * _CG, N - _CG)
                pltpu.sync_copy(gate_hbm.at[pl.ds(b, _CG)], gbig_v.at[pl.ds(0, _CG)])
                lo = jnp.maximum(s, c0 + k * _CG)
                hi = jnp.minimum(e, c0 + k * _CG + _CG)

                def grp(j, mv):
                    idx = b + j * _L + lane
                    v = gbig_v[pl.ds(j * _L, _L)]
                    msk = (idx >= lo) & (idx < hi)
                    return jnp.maximum(mv, jnp.where(msk, v, -jnp.inf))

                return lax.fori_loop(0, _CG // _L, grp, m_vec)

            m_vec = lax.fori_loop(0, nch_g, max_chunk,
                                  jnp.full((_L,), -jnp.inf, dtype=jnp.float32))
            m = _red16(m_vec, jnp.maximum)

            def den_chunk(k, d_vec):
                b = jnp.minimum(c0 + k * _CG, N - _CG)
                pltpu.sync_copy(gate_hbm.at[pl.ds(b, _CG)], gbig_v.at[pl.ds(0, _CG)])
                lo = jnp.maximum(s, c0 + k * _CG)
                hi = jnp.minimum(e, c0 + k * _CG + _CG)

                def grp(j, dv):
                    idx = b + j * _L + lane
                    v = gbig_v[pl.ds(j * _L, _L)]
                    msk = (idx >= lo) & (idx < hi)
                    return dv + jnp.where(msk, jnp.exp(v - m), 0.0)

                return lax.fori_loop(0, _CG // _L, grp, d_vec)

            d_vec = lax.fori_loop(0, nch_g, den_chunk,
                                  jnp.zeros((_L,), dtype=jnp.float32))
            dinv = _vrecip(_red16(d_vec, jnp.add) + 1e-16)

            nch_x = (e - c0 + _CX - 1) // _CX

            def x_chunk(k, acc):
                b = jnp.minimum(c0 + k * _CX, N - _CX)
                pltpu.sync_copy(gate_hbm.at[pl.ds(b, _CX)], gbig_v.at[pl.ds(0, _CX)])
                pltpu.sync_copy(x_hbm.at[pl.ds(b, _CX)], x0_v)
                lo = jnp.maximum(s, c0 + k * _CX)
                hi = jnp.minimum(e, c0 + k * _CX + _CX)

                def grp(j, acc_in):
                    idx = b + j * _L + lane
                    v = gbig_v[pl.ds(j * _L, _L)]
                    msk = (idx >= lo) & (idx < hi)
                    a = jnp.where(msk, jnp.exp(v - m), 0.0) * dinv
                    acc_out = list(acc_in)
                    for r in range(_L):
                        ar = a[r]
                        for c in range(D // _L):
                            acc_out[c] = acc_out[c] + ar * x0_v[j * _L + r, pl.ds(c * _L, _L)]
                    return tuple(acc_out)

                return lax.fori_loop(0, _CX // _L, grp, acc)

            acc = lax.fori_loop(0, nch_x, x_chunk, acc0)
            for c in range(D // _L):
                acc_area[sloc, pl.ds(c * _L, _L)] = acc[c]
            return 0

        lax.fori_loop(0, _SEG_PER_W, seg_body, 0)
        return 0

    lax.cond(fits, fast_path, slow_path)

    for sloc in range(_SEG_PER_W):
        pltpu.async_copy(acc_area.at[sloc], out_hbm.at[g0 + sloc], semx0)
    for sloc in range(_SEG_PER_W):
        pltpu.make_async_copy(acc_area.at[sloc], out_hbm.at[g0 + sloc], semx0).wait()


_seg_kernel = functools.partial(
    pl.kernel,
    out_type=jax.ShapeDtypeStruct((G, D), jnp.float32),
    mesh=plsc.VectorSubcoreMesh(core_axis_name="c", subcore_axis_name="s"),
    scratch_types=[
        pltpu.VMEM((_L,), jnp.int32),
        pltpu.VMEM((_CAPW + _L,), jnp.float32),
        pltpu.VMEM((_CAPW + _L,), jnp.float32),
        pltpu.VMEM((_CX, D), jnp.float32),
        pltpu.VMEM((_CX, D), jnp.float32),
        pltpu.VMEM((_SEG_PER_W, D), jnp.float32),
        pltpu.VMEM((_SEG_PER_W * _L,), jnp.float32),
        pltpu.SemaphoreType.DMA,
        pltpu.SemaphoreType.DMA,
        pltpu.SemaphoreType.DMA,
    ],
)(_seg_kernel_body)


def kernel(x, batch, W1, b1, W2, b2):
    batch32 = batch.astype(jnp.int32)
    gate = _gate_mlp(x, W1, b1.reshape(1, H), W2.reshape(1, H), b2.reshape(1, 1))
    gate1 = gate.reshape(N)
    return _seg_kernel(gate1, batch32, x)


# speedup vs baseline: 1.1680x; 1.1680x over previous
import functools

import jax
import jax.numpy as jnp
from jax import lax
from jax.experimental import pallas as pl
from jax.experimental.pallas import tpu as pltpu
from jax.experimental.pallas import tpu_sc as plsc

N = 100000
D = 128
H = 64
G = 256


BLK = 20000


def _gate_body(x_ref, w1_ref, b1_ref, w2_ref, b2_ref, gate_ref):
    h = jnp.dot(x_ref[...], w1_ref[...], preferred_element_type=jnp.float32)
    h = jnp.maximum(h + b1_ref[...], 0.0)
    gate_ref[...] = jnp.sum(h * w2_ref[...], axis=1, keepdims=True) + b2_ref[...]


def _gate_mlp(x, W1, b1r, w2r, b2r):
    return pl.pallas_call(
        _gate_body,
        grid=(N // BLK,),
        in_specs=[
            pl.BlockSpec((BLK, D), lambda i: (i, 0)),
            pl.BlockSpec((D, H), lambda i: (0, 0)),
            pl.BlockSpec((1, H), lambda i: (0, 0)),
            pl.BlockSpec((1, H), lambda i: (0, 0)),
            pl.BlockSpec((1, 1), lambda i: (0, 0)),
        ],
        out_specs=pl.BlockSpec((BLK, 1), lambda i: (i, 0)),
        out_shape=jax.ShapeDtypeStruct((N, 1), jnp.float32),
    )(x, W1, b1r, w2r, b2r)


_NC = 2
_NS = 16
_L = 16
_NW = _NC * _NS
_SEG_PER_W = G // _NW
_CAPW = 4096
_CG = 512
_CX = 384


def _red16(v, op):
    r = v[0]
    for i in range(1, _L):
        r = op(r, v[i])
    return r


def _vrecip(den):
    return jnp.ones((_L,), dtype=jnp.float32) / (
        jnp.zeros((_L,), dtype=jnp.float32) + den)


def _seg_kernel_body(gate_hbm, batch_hbm, x_hbm, out_hbm,
                     vals_v, gbig_v, e_v, x0_v, x1_v, acc_area, dinv_v,
                     sem, semx0, semx1):
    wid = lax.axis_index("s") * _NC + lax.axis_index("c")
    lane = lax.iota(jnp.int32, _L)
    g0 = wid * _SEG_PER_W

    q = g0 + lane

    def bs_step(t, lohi):
        lo, hi = lohi
        mid = (lo + hi) >> 1
        pltpu.async_copy(batch_hbm.at[jnp.minimum(mid, N - 1)], vals_v, sem).wait()
        lt = vals_v[...] < q
        upd = lo < hi
        lo2 = jnp.where(upd, jnp.where(lt, mid + 1, lo), lo)
        hi2 = jnp.where(upd, jnp.where(lt, hi, mid), hi)
        return (lo2, hi2)

    swin, _ = lax.fori_loop(0, 17, bs_step,
                            (jnp.zeros((_L,), jnp.int32),
                             jnp.full((_L,), N, dtype=jnp.int32)))
    sb = [swin[i] for i in range(_SEG_PER_W + 1)]
    w_s, w_e = sb[0], sb[_SEG_PER_W]
    w_b0 = jnp.minimum((w_s // 8) * 8, N - _CAPW)
    fits = (w_e - w_b0) <= _CAPW

    acc0 = tuple(jnp.zeros((_L,), dtype=jnp.float32) for _ in range(D // _L))
    zero16 = jnp.zeros((_L,), dtype=jnp.float32)

    def fast_path():
        def xbase(k):
            return jnp.minimum(w_b0 + k * _CX, N - _CX)

        pltpu.async_copy(x_hbm.at[pl.ds(xbase(0), _CX)], x0_v, semx0)
        pltpu.async_copy(gate_hbm.at[pl.ds(w_b0, _CAPW)], gbig_v.at[pl.ds(0, _CAPW)], semx1)
        nwg = (w_e - w_b0 + _L - 1) // _L

        def zero_grp(j, _):
            e_v[pl.ds(j * _L, _L)] = zero16
            return 0

        lax.fori_loop(0, nwg, zero_grp, 0)
        pltpu.make_async_copy(gate_hbm.at[pl.ds(0, _CAPW)], gbig_v.at[pl.ds(0, _CAPW)], semx1).wait()

        for sloc in range(_SEG_PER_W):
            s_g, e_g = sb[sloc], sb[sloc + 1]
            j0 = (s_g - w_b0) // _L
            j1 = (e_g - w_b0 + _L - 1) // _L

            def grp0(j, mv, s_g=s_g, e_g=e_g):
                idx = w_b0 + j * _L + lane
                v = gbig_v[pl.ds(j * _L, _L)]
                msk = (idx >= s_g) & (idx < e_g)
                return jnp.maximum(mv, jnp.where(msk, v, -jnp.inf))

            m_vec = lax.fori_loop(j0, j1, grp0,
                                  jnp.full((_L,), -jnp.inf, dtype=jnp.float32))
            m = _red16(m_vec, jnp.maximum)

            def grp1(j, dv, s_g=s_g, e_g=e_g, m=m):
                idx = w_b0 + j * _L + lane
                v = gbig_v[pl.ds(j * _L, _L)]
                msk = (idx >= s_g) & (idx < e_g)
                ev = jnp.where(msk, jnp.exp(v - m), 0.0)
                e_v[pl.ds(j * _L, _L)] = e_v[pl.ds(j * _L, _L)] + ev
                return dv + ev

            d_vec = lax.fori_loop(j0, j1, grp1, jnp.zeros((_L,), dtype=jnp.float32))
            dinv_v[pl.ds(sloc * _L, _L)] = _vrecip(_red16(d_vec, jnp.add) + 1e-16)

        for sloc in range(_SEG_PER_W):
            for c in range(D // _L):
                acc_area[sloc, pl.ds(c * _L, _L)] = zero16

        nch = (w_e - w_b0 + _CX - 1) // _CX
        npairs = (nch + 1) // 2

        def process_chunk(k, xbuf):
            b = xbase(k)
            rel = b - w_b0
            for sloc in range(_SEG_PER_W):
                s_g, e_g = sb[sloc], sb[sloc + 1]
                glo = jnp.maximum(s_g, b)
                ghi = jnp.minimum(e_g, b + _CX)
                cj0 = jnp.maximum(glo - b, 0) // _L
                cj1 = (jnp.maximum(ghi - b, 0) + _L - 1) // _L
                dinv_g = dinv_v[pl.ds(sloc * _L, _L)]

                def grp(j, accs, s_g=s_g, e_g=e_g, b=b, rel=rel, dinv_g=dinv_g):
                    idx = b + j * _L + lane
                    ev = e_v[pl.ds(rel + j * _L, _L)]
                    msk = (idx >= s_g) & (idx < e_g)
                    a = jnp.where(msk, ev, 0.0) * dinv_g
                    acc_out = list(accs)
                    for r in range(_L):
                        ar = a[r]
                        for c in range(D // _L):
                            acc_out[c] = acc_out[c] + ar * xbuf[j * _L + r, pl.ds(c * _L, _L)]
                    return tuple(acc_out)

                accs_in = tuple(acc_area[sloc, pl.ds(c * _L, _L)] for c in range(D // _L))
                accs = lax.fori_loop(cj0, cj1, grp, accs_in)
                for c in range(D // _L):
                    acc_area[sloc, pl.ds(c * _L, _L)] = accs[c]

        def pair_body(p, _):
            pltpu.async_copy(x_hbm.at[pl.ds(xbase(2 * p + 1), _CX)], x1_v, semx1)
            pltpu.make_async_copy(x_hbm.at[pl.ds(0, _CX)], x0_v, semx0).wait()
            process_chunk(2 * p, x0_v)
            pltpu.async_copy(x_hbm.at[pl.ds(xbase(2 * p + 2), _CX)], x0_v, semx0)
            pltpu.make_async_copy(x_hbm.at[pl.ds(0, _CX)], x1_v, semx1).wait()
            process_chunk(2 * p + 1, x1_v)
            return 0

        lax.fori_loop(0, npairs, pair_body, 0)
        pltpu.make_async_copy(x_hbm.at[pl.ds(0, _CX)], x0_v, semx0).wait()
        return 0

    def slow_path():
        def seg_body(sloc, _):
            s = _red16(jnp.where(lane == sloc, swin, 0), jnp.add)
            e = _red16(jnp.where(lane == sloc + 1, swin, 0), jnp.add)
            c0 = (s // 8) * 8
            nch_g = (e - c0 + _CG - 1) // _CG

            def max_chunk(k, m_vec):
                b = jnp.minimum(c0 + k * _CG, N - _CG)
                pltpu.sync_copy(gate_hbm.at[pl.ds(b, _CG)], gbig_v.at[pl.ds(0, _CG)])
                lo = jnp.maximum(s, c0 + k * _CG)
                hi = jnp.minimum(e, c0 + k * _CG + _CG)

                def grp(j, mv):
                    idx = b + j * _L + lane
                    v = gbig_v[pl.ds(j * _L, _L)]
                    msk = (idx >= lo) & (idx < hi)
                    return jnp.maximum(mv, jnp.where(msk, v, -jnp.inf))

                return lax.fori_loop(0, _CG // _L, grp, m_vec)

            m_vec = lax.fori_loop(0, nch_g, max_chunk,
                                  jnp.full((_L,), -jnp.inf, dtype=jnp.float32))
            m = _red16(m_vec, jnp.maximum)

            def den_chunk(k, d_vec):
                b = jnp.minimum(c0 + k * _CG, N - _CG)
                pltpu.sync_copy(gate_hbm.at[pl.ds(b, _CG)], gbig_v.at[pl.ds(0, _CG)])
                lo = jnp.maximum(s, c0 + k * _CG)
                hi = jnp.minimum(e, c0 + k * _CG + _CG)

                def grp(j, dv):
                    idx = b + j * _L + lane
                    v = gbig_v[pl.ds(j * _L, _L)]
                    msk = (idx >= lo) & (idx < hi)
                    return dv + jnp.where(msk, jnp.exp(v - m), 0.0)

                return lax.fori_loop(0, _CG // _L, grp, d_vec)

            d_vec = lax.fori_loop(0, nch_g, den_chunk,
                                  jnp.zeros((_L,), dtype=jnp.float32))
            dinv = _vrecip(_red16(d_vec, jnp.add) + 1e-16)

            nch_x = (e - c0 + _CX - 1) // _CX

            def x_chunk(k, acc):
                b = jnp.minimum(c0 + k * _CX, N - _CX)
                pltpu.sync_copy(gate_hbm.at[pl.ds(b, _CX)], gbig_v.at[pl.ds(0, _CX)])
                pltpu.sync_copy(x_hbm.at[pl.ds(b, _CX)], x0_v)
                lo = jnp.maximum(s, c0 + k * _CX)
                hi = jnp.minimum(e, c0 + k * _CX + _CX)

                def grp(j, acc_in):
                    idx = b + j * _L + lane
                    v = gbig_v[pl.ds(j * _L, _L)]
                    msk = (idx >= lo) & (idx < hi)
                    a = jnp.where(msk, jnp.exp(v - m), 0.0) * dinv
                    acc_out = list(acc_in)
                    for r in range(_L):
                        ar = a[r]
                        for c in range(D // _L):
                            acc_out[c] = acc_out[c] + ar * x0_v[j * _L + r, pl.ds(c * _L, _L)]
                    return tuple(acc_out)

                return lax.fori_loop(0, _CX // _L, grp, acc)

            acc = lax.fori_loop(0, nch_x, x_chunk, acc0)
            for c in range(D // _L):
                acc_area[sloc, pl.ds(c * _L, _L)] = acc[c]
            return 0

        lax.fori_loop(0, _SEG_PER_W, seg_body, 0)
        return 0

    lax.cond(fits, fast_path, slow_path)

    for sloc in range(_SEG_PER_W):
        pltpu.async_copy(acc_area.at[sloc], out_hbm.at[g0 + sloc], semx0)
    for sloc in range(_SEG_PER_W):
        pltpu.make_async_copy(acc_area.at[sloc], out_hbm.at[g0 + sloc], semx0).wait()


_seg_kernel = functools.partial(
    pl.kernel,
    out_type=jax.ShapeDtypeStruct((G, D), jnp.float32),
    mesh=plsc.VectorSubcoreMesh(core_axis_name="c", subcore_axis_name="s"),
    scratch_types=[
        pltpu.VMEM((_L,), jnp.int32),
        pltpu.VMEM((_CAPW + _L,), jnp.float32),
        pltpu.VMEM((_CAPW + _L,), jnp.float32),
        pltpu.VMEM((_CX, D), jnp.float32),
        pltpu.VMEM((_CX, D), jnp.float32),
        pltpu.VMEM((_SEG_PER_W, D), jnp.float32),
        pltpu.VMEM((_SEG_PER_W * _L,), jnp.float32),
        pltpu.SemaphoreType.DMA,
        pltpu.SemaphoreType.DMA,
        pltpu.SemaphoreType.DMA,
    ],
)(_seg_kernel_body)


def kernel(x, batch, W1, b1, W2, b2):
    batch32 = batch.astype(jnp.int32)
    gate = _gate_mlp(x, W1, b1.reshape(1, H), W2.reshape(1, H), b2.reshape(1, 1))
    gate1 = gate.reshape(N)
    return _seg_kernel(gate1, batch32, x)
